# baseline (speedup 1.0000x reference)
import functools

import jax
import jax.numpy as jnp
from jax import lax
from jax.experimental import pallas as pl
from jax.experimental.pallas import tpu as pltpu
from jax.experimental.pallas import tpu_sc as plsc

NUM_NODES = 1000000
EMBED_DIM = 64
BATCH = 16384

_info = plsc.get_sparse_core_info()
_NC, _NS = _info.num_cores, _info.num_subcores
_NW = _NC * _NS
_B_PER_W = BATCH // _NW
_SLAB = 96
_NT = NUM_NODES // 8
_GENS = _NT // (_NW * _SLAB)

_mesh = plsc.VectorSubcoreMesh(core_axis_name="c", subcore_axis_name="s")


@functools.partial(
    pl.kernel,
    mesh=_mesh,
    out_type=jax.ShapeDtypeStruct((BATCH, EMBED_DIM), jnp.float32),
    scratch_types=[
        pltpu.VMEM((_SLAB, 8, EMBED_DIM), jnp.float32),
        pltpu.SemaphoreType.DMA,
    ],
    compiler_params=pltpu.CompilerParams(needs_layout_passes=False),
)
def _sweep_kernel(idx_hbm, table_hbm, out_hbm, slab_v, sem):
    wid = lax.axis_index("s") * _NC + lax.axis_index("c")
    base = wid * _B_PER_W

    table_view = table_hbm.reshape(_NT, 8, EMBED_DIM)

    def gen_body(g, carry):
        lo = (g * _NW + wid) * _SLAB
        pltpu.sync_copy(table_view.at[pl.ds(lo, _SLAB)], slab_v)
        return carry

    lax.fori_loop(0, _GENS, gen_body, 0)
    out_view = slab_v.reshape(_SLAB * 8, EMBED_DIM)
    pltpu.sync_copy(
        out_view.at[pl.ds(0, _B_PER_W)], out_hbm.at[pl.ds(base, _B_PER_W)]
    )


def kernel(indices, weight):
    idx = indices.astype(jnp.int32)
    return _sweep_kernel(idx, weight)


# speedup vs baseline: 1.4265x; 1.4265x over previous
import functools

import jax
import jax.numpy as jnp
from jax import lax
from jax.experimental import pallas as pl
from jax.experimental.pallas import tpu as pltpu
from jax.experimental.pallas import tpu_sc as plsc

NUM_NODES = 1000000
EMBED_DIM = 64
BATCH = 16384

_info = plsc.get_sparse_core_info()
_NC, _NS = _info.num_cores, _info.num_subcores
_NW = _NC * _NS
_B_PER_W = BATCH // _NW
_NSTREAM = 256
_NDMA = _B_PER_W - _NSTREAM

_mesh = plsc.VectorSubcoreMesh(core_axis_name="c", subcore_axis_name="s")


@functools.partial(
    pl.kernel,
    mesh=_mesh,
    out_type=jax.ShapeDtypeStruct((BATCH, EMBED_DIM), jnp.float32),
    scratch_types=[
        pltpu.VMEM((_B_PER_W,), jnp.int32),
        pltpu.VMEM((_NSTREAM, EMBED_DIM), jnp.float32),
        pltpu.VMEM_SHARED((_NS, _NDMA, EMBED_DIM), jnp.float32),
        pltpu.SemaphoreType.DMA,
        pltpu.SemaphoreType.DMA,
    ],
)
def _gather_kernel(idx_hbm, table_hbm, out_hbm, idx_v, rows_v, rows_s, sem_a, sem_b):
    sid = lax.axis_index("s")
    wid = sid * _NC + lax.axis_index("c")
    base = wid * _B_PER_W
    pltpu.sync_copy(idx_hbm.at[pl.ds(base, _B_PER_W)], idx_v)
    my_s = rows_s.at[sid]

    def fire_stream(g, carry):
        v = idx_v[pl.ds(g * 16, 16)]
        for l in range(16):
            pltpu.make_async_copy(
                table_hbm.at[pl.ds(v[l], 1)],
                rows_v.at[pl.ds(g * 16 + l, 1)],
                sem_a,
            ).start()
        return carry

    def fire_dma(g, carry):
        v = idx_v[pl.ds(_NSTREAM + g * 16, 16)]
        for l in range(16):
            pltpu.make_async_copy(
                table_hbm.at[pl.ds(v[l], 1)],
                my_s.at[pl.ds(g * 16 + l, 1)],
                sem_b,
            ).start()
        return carry

    def drain_stream(g, carry):
        for l in range(16):
            pltpu.make_async_copy(
                table_hbm.at[pl.ds(0, 1)],
                rows_v.at[pl.ds(g * 16 + l, 1)],
                sem_a,
            ).wait()
        return carry

    def drain_dma(g, carry):
        for l in range(16):
            pltpu.make_async_copy(
                table_hbm.at[pl.ds(0, 1)],
                my_s.at[pl.ds(g * 16 + l, 1)],
                sem_b,
            ).wait()
        return carry

    lax.fori_loop(0, _NDMA // 16, fire_dma, 0)
    lax.fori_loop(0, _NSTREAM // 16, fire_stream, 0)
    lax.fori_loop(0, _NSTREAM // 16, drain_stream, 0)
    lax.fori_loop(0, _NDMA // 16, drain_dma, 0)
    pltpu.sync_copy(rows_v, out_hbm.at[pl.ds(base, _NSTREAM)])
    pltpu.sync_copy(my_s, out_hbm.at[pl.ds(base + _NSTREAM, _NDMA)])


def kernel(indices, weight):
    idx = indices.astype(jnp.int32)
    return _gather_kernel(idx, weight)
